# baseline (speedup 1.0000x reference)
import functools

import jax
import jax.numpy as jnp
from jax import lax
from jax.experimental import pallas as pl
from jax.experimental.pallas import tpu as pltpu
from jax.experimental.pallas import tpu_sc as plsc


def _tc_add_kernel(x_ref, pe_ref, o_ref):
    o_ref[...] = x_ref[...] + pe_ref[...]


@jax.jit
def _kernel_tc(x, pe_table):
    B, S, D = x.shape
    R = 2048

    grid = (S // R, B)

    return pl.pallas_call(
        _tc_add_kernel,
        grid=grid,
        in_specs=[
            pl.BlockSpec((1, R, D), lambda i, j: (j, i, 0)),
            pl.BlockSpec((R, D), lambda i, j: (i, 0)),
        ],
        out_specs=pl.BlockSpec((1, R, D), lambda i, j: (j, i, 0)),
        out_shape=jax.ShapeDtypeStruct((B, S, D), x.dtype),
        compiler_params=pltpu.CompilerParams(
            dimension_semantics=("arbitrary", "arbitrary"),
        ),
    )(x, pe_table)


_NW = 32
_ROWS_PER_W = 64
_XB = 32
_D = 1024


def _make_sc_add(B, S, D):
    CH = _XB * D
    pe_w = _ROWS_PER_W * D
    batch_elems = S * D
    n_pe_chunks = _ROWS_PER_W // _XB

    mesh = plsc.VectorSubcoreMesh(core_axis_name="c", subcore_axis_name="s")

    @functools.partial(
        pl.kernel,
        mesh=mesh,
        out_type=jax.ShapeDtypeStruct((B * S * D,), jnp.float32),
        scratch_types=[
            pltpu.VMEM((CH,), jnp.float32),
            pltpu.VMEM((CH,), jnp.float32),
        ],
    )
    def sc_add(x_hbm, pe_hbm, out_hbm, x_v, pe_v):
        c = lax.axis_index("c")
        s = lax.axis_index("s")
        wid = s * 2 + c
        pe_base = wid * pe_w
        for pc in range(n_pe_chunks):
            pltpu.sync_copy(pe_hbm.at[pl.ds(pe_base + pc * CH, CH)], pe_v)
            for b in range(B):
                xs = b * batch_elems + pe_base + pc * CH
                pltpu.sync_copy(x_hbm.at[pl.ds(xs, CH)], x_v)

                def body(i, _):
                    sl = pl.ds(i * 16, 16)
                    x_v[sl] = x_v[sl] + pe_v[sl]
                    return 0

                lax.fori_loop(0, CH // 16, body, 0)
                pltpu.sync_copy(x_v, out_hbm.at[pl.ds(xs, CH)])

    return sc_add


@jax.jit
def _kernel_sc(x, pe_table):
    B, S, D = x.shape
    out = _make_sc_add(B, S, D)(x.reshape(-1), pe_table.reshape(-1))
    return out.reshape(B, S, D)


kernel = _kernel_sc


# speedup vs baseline: 1.4368x; 1.4368x over previous
import functools

import jax
import jax.numpy as jnp
from jax import lax
from jax.experimental import pallas as pl
from jax.experimental.pallas import tpu as pltpu
from jax.experimental.pallas import tpu_sc as plsc


def _tc_add_kernel(x_ref, pe_ref, o_ref):
    o_ref[...] = x_ref[...] + pe_ref[...]


@jax.jit
def _kernel_tc(x, pe_table):
    B, S, D = x.shape
    R = 2048

    grid = (S // R, B)

    return pl.pallas_call(
        _tc_add_kernel,
        grid=grid,
        in_specs=[
            pl.BlockSpec((1, R, D), lambda i, j: (j, i, 0)),
            pl.BlockSpec((R, D), lambda i, j: (i, 0)),
        ],
        out_specs=pl.BlockSpec((1, R, D), lambda i, j: (j, i, 0)),
        out_shape=jax.ShapeDtypeStruct((B, S, D), x.dtype),
        compiler_params=pltpu.CompilerParams(
            dimension_semantics=("arbitrary", "arbitrary"),
        ),
    )(x, pe_table)


_NW = 32
_ROWS_PER_W = 64
_XB = 16


def _make_sc_add(B, S, D):
    CH = _XB * D
    pe_w = _ROWS_PER_W * D
    batch_elems = S * D
    n_pe_chunks = _ROWS_PER_W // _XB
    n_chunks = n_pe_chunks * B

    mesh = plsc.VectorSubcoreMesh(core_axis_name="c", subcore_axis_name="s")

    @functools.partial(
        pl.kernel,
        mesh=mesh,
        out_type=jax.ShapeDtypeStruct((B * S * D,), jnp.float32),
        scratch_types=[
            pltpu.VMEM((2, CH), jnp.float32),
            pltpu.VMEM((2, CH), jnp.float32),
            pltpu.VMEM((2, CH), jnp.float32),
            pltpu.SemaphoreType.DMA((2,)),
            pltpu.SemaphoreType.DMA((2,)),
            pltpu.SemaphoreType.DMA((2,)),
        ],
    )
    def sc_add(x_hbm, pe_hbm, out_hbm, x_v, o_v, pe_v, in_sem, out_sem, pe_sem):
        c = lax.axis_index("c")
        s = lax.axis_index("s")
        wid = s * 2 + c
        pe_base = wid * pe_w

        def xs(k):
            p, b = divmod(k, B)
            return b * batch_elems + pe_base + p * CH

        def start_in(k):
            return pltpu.async_copy(
                x_hbm.at[pl.ds(xs(k), CH)], x_v.at[k % 2], in_sem.at[k % 2])

        def start_pe(p):
            return pltpu.async_copy(
                pe_hbm.at[pl.ds(pe_base + p * CH, CH)], pe_v.at[p % 2],
                pe_sem.at[p % 2])

        in_cp = {k: start_in(k) for k in range(2)}
        pe_cp = {p: start_pe(p) for p in range(2)}
        out_cp = {}

        for k in range(n_chunks):
            p = k // B
            if k % B == 0:
                pe_cp[p].wait()
            in_cp[k].wait()
            if k - 2 in out_cp:
                out_cp[k - 2].wait()

            xb, ob, pb = x_v.at[k % 2], o_v.at[k % 2], pe_v.at[p % 2]

            @plsc.parallel_loop(0, CH, 16, unroll=8)
            def _(i):
                sl = pl.ds(i, 16)
                ob[sl] = xb[sl] + pb[sl]

            out_cp[k] = pltpu.async_copy(
                ob, out_hbm.at[pl.ds(xs(k), CH)], out_sem.at[k % 2])
            if k + 2 < n_chunks:
                in_cp[k + 2] = start_in(k + 2)
            if k % B == B - 1 and p + 2 < n_pe_chunks:
                pe_cp[p + 2] = start_pe(p + 2)

        out_cp[n_chunks - 2].wait()
        out_cp[n_chunks - 1].wait()

    return sc_add


@jax.jit
def _kernel_sc(x, pe_table):
    B, S, D = x.shape
    out = _make_sc_add(B, S, D)(x.reshape(-1), pe_table.reshape(-1))
    return out.reshape(B, S, D)


kernel = _kernel_sc


# speedup vs baseline: 3.8833x; 2.7027x over previous
import functools

import jax
import jax.numpy as jnp
from jax import lax
from jax.experimental import pallas as pl
from jax.experimental.pallas import tpu as pltpu
from jax.experimental.pallas import tpu_sc as plsc


def _tc_add_kernel(x_ref, pe_ref, o_ref):
    o_ref[...] = x_ref[...] + pe_ref[...]


@jax.jit
def _kernel_tc(x, pe_table):
    B, S, D = x.shape
    R = 2048

    grid = (S // R, B)

    return pl.pallas_call(
        _tc_add_kernel,
        grid=grid,
        in_specs=[
            pl.BlockSpec((1, R, D), lambda i, j: (j, i, 0)),
            pl.BlockSpec((R, D), lambda i, j: (i, 0)),
        ],
        out_specs=pl.BlockSpec((1, R, D), lambda i, j: (j, i, 0)),
        out_shape=jax.ShapeDtypeStruct((B, S, D), x.dtype),
        compiler_params=pltpu.CompilerParams(
            dimension_semantics=("arbitrary", "arbitrary"),
        ),
    )(x, pe_table)


_NW = 32
_ROWS_PER_W = 64
_XB = 16


def _make_sc_add(B, S, D):
    n_pe_chunks = _ROWS_PER_W // _XB
    n_chunks = n_pe_chunks * B

    mesh = plsc.VectorSubcoreMesh(core_axis_name="c", subcore_axis_name="s")

    @functools.partial(
        pl.kernel,
        mesh=mesh,
        out_type=jax.ShapeDtypeStruct((B * S, D), jnp.float32),
        scratch_types=[
            pltpu.VMEM((2, _XB, D), jnp.float32),
            pltpu.VMEM((2, _XB, D), jnp.float32),
            pltpu.VMEM((2, _XB, D), jnp.float32),
            pltpu.SemaphoreType.DMA((2,)),
            pltpu.SemaphoreType.DMA((2,)),
            pltpu.SemaphoreType.DMA((2,)),
        ],
    )
    def sc_add(x_hbm, pe_hbm, out_hbm, x_v, o_v, pe_v, in_sem, out_sem, pe_sem):
        c = lax.axis_index("c")
        s = lax.axis_index("s")
        wid = s * 2 + c
        pe_row0 = wid * _ROWS_PER_W

        def row0(k):
            p, b = divmod(k, B)
            return b * S + pe_row0 + p * _XB

        def start_in(k):
            return pltpu.async_copy(
                x_hbm.at[pl.ds(row0(k), _XB)], x_v.at[k % 2], in_sem.at[k % 2])

        def start_pe(p):
            return pltpu.async_copy(
                pe_hbm.at[pl.ds(pe_row0 + p * _XB, _XB)], pe_v.at[p % 2],
                pe_sem.at[p % 2])

        in_cp = {k: start_in(k) for k in range(2)}
        pe_cp = {p: start_pe(p) for p in range(2)}
        out_cp = {}

        for k in range(n_chunks):
            p = k // B
            if k % B == 0:
                pe_cp[p].wait()
            in_cp[k].wait()
            if k - 2 in out_cp:
                out_cp[k - 2].wait()

            xb, ob, pb = x_v.at[k % 2], o_v.at[k % 2], pe_v.at[p % 2]

            @plsc.parallel_loop(0, _XB * D, 16, unroll=8)
            def _(i):
                r = i // D
                col = i - r * D
                sl = pl.ds(col, 16)
                ob[r, sl] = xb[r, sl] + pb[r, sl]

            out_cp[k] = pltpu.async_copy(
                ob, out_hbm.at[pl.ds(row0(k), _XB)], out_sem.at[k % 2])
            if k + 2 < n_chunks:
                in_cp[k + 2] = start_in(k + 2)
            if k % B == B - 1 and p + 2 < n_pe_chunks:
                pe_cp[p + 2] = start_pe(p + 2)

        out_cp[n_chunks - 2].wait()
        out_cp[n_chunks - 1].wait()

    return sc_add


@jax.jit
def _kernel_sc(x, pe_table):
    B, S, D = x.shape
    out = _make_sc_add(B, S, D)(x.reshape(B * S, D), pe_table)
    return out.reshape(B, S, D)


kernel = _kernel_sc
